# baseline (speedup 1.0000x reference)
import jax
import jax.numpy as jnp
from jax import lax
from jax.experimental import pallas as pl
from jax.experimental.pallas import tpu as pltpu
from jax.experimental.pallas import tpu_sc as plsc

_BATCH = 16384
_NUM_FIELDS = 26
_VOCAB = 100000
_EMB_DIM = 32

_NC = 2
_C = 50048
_CB = _VOCAB - _C
_M = 12288
_T = 2048


def _sc_body(x_hbm, table_hbm, out_hbm, buf_a, buf_b, buf_t, idx_m, idx_t,
             acc, sem_a, sem_b, sem_t, sem_im, sem_it):
    t = lax.axis_index("s")
    c = lax.axis_index("c")
    s = t * _NC + c
    zeros16 = jnp.zeros((16,), jnp.float32)

    def stage_a(i):
        pltpu.async_copy(
            table_hbm.at[i, s, pl.ds(0, _C)], buf_a.at[pl.ds(0, _C)], sem_a)

    def stage_b(i):
        pltpu.async_copy(
            table_hbm.at[i, s, pl.ds(_C, _CB - 32)],
            buf_b.at[pl.ds(0, _CB - 32)], sem_b)
        pltpu.async_copy(
            table_hbm.at[i, s, pl.ds(_VOCAB - 32, 32)], buf_t, sem_t)

    def drain_a():
        pltpu.make_async_copy(
            table_hbm.at[0, 0, pl.ds(0, _C)], buf_a.at[pl.ds(0, _C)],
            sem_a).wait()

    def drain_b():
        pltpu.make_async_copy(
            table_hbm.at[0, 0, pl.ds(_C, _CB - 32)],
            buf_b.at[pl.ds(0, _CB - 32)], sem_b).wait()
        pltpu.make_async_copy(
            table_hbm.at[0, 0, pl.ds(_VOCAB - 32, 32)], buf_t, sem_t).wait()
        buf_b[pl.ds(_CB - 32, 16)] = buf_t[pl.ds(0, 16)]
        buf_b[pl.ds(_CB - 16, 16)] = buf_t[pl.ds(16, 16)]

    def stage_m(i):
        pltpu.async_copy(x_hbm.at[i, pl.ds(0, _M)], idx_m, sem_im)

    def drain_m():
        pltpu.make_async_copy(
            x_hbm.at[0, pl.ds(0, _M)], idx_m, sem_im).wait()

    def stage_t_idx(i, k):
        pltpu.async_copy(
            x_hbm.at[i, pl.ds(_M + k * _T, _T)], idx_t, sem_it)

    def drain_t_idx():
        pltpu.make_async_copy(
            x_hbm.at[0, pl.ds(0, _T)], idx_t, sem_it).wait()

    def chunks(ibuf, nchunks, acc_base, in_a, first):
        def chunk(j, carry):
            v = ibuf[pl.ds(j * 16, 16)]
            if in_a:
                g = plsc.load_gather(buf_a, [jnp.minimum(v, _C)])
            else:
                vb = plsc.bitcast(
                    jnp.minimum(
                        plsc.bitcast(v - _C, jnp.uint32), jnp.uint32(_CB)),
                    jnp.int32)
                g = plsc.load_gather(buf_b, [vb])
            sl = pl.ds(acc_base + j * 16, 16)
            if first:
                acc[sl] = g
            else:
                plsc.addupdate(acc.at[sl], g)
            return carry
        lax.fori_loop(0, nchunks, chunk, 0, unroll=8)

    def run_pass(i, in_a, first):
        stage_t_idx(i, 0)
        chunks(idx_m, _M // 16, 0, in_a, first)
        drain_t_idx()
        chunks(idx_t, _T // 16, _M, in_a, first)
        stage_t_idx(i, 1)
        drain_t_idx()
        chunks(idx_t, _T // 16, _M + _T, in_a, first)

    buf_a[pl.ds(_C, 16)] = zeros16
    buf_b[pl.ds(_CB, 16)] = zeros16

    stage_a(0)
    stage_b(0)

    def field(i, first):
        nxt = jnp.minimum(i + 1, _NUM_FIELDS - 1)
        stage_m(i)
        drain_a()
        drain_m()
        run_pass(i, True, first)
        stage_a(nxt)
        drain_b()
        run_pass(i, False, False)
        stage_b(nxt)

    field(0, True)
    lax.fori_loop(1, _NUM_FIELDS, lambda i, cr: (field(i, False), cr)[1], 0)
    drain_a()
    drain_b()
    pltpu.sync_copy(acc, out_hbm.at[s])


def kernel(x, tables):
    x_t = x.T
    tables_t = tables.transpose(0, 2, 1)

    f = pl.kernel(
        _sc_body,
        out_type=jax.ShapeDtypeStruct((_EMB_DIM, _BATCH), jnp.float32),
        mesh=plsc.VectorSubcoreMesh(core_axis_name="c", subcore_axis_name="s"),
        scratch_types=[
            pltpu.VMEM((_C + 16,), jnp.float32),
            pltpu.VMEM((_CB + 16,), jnp.float32),
            pltpu.VMEM((32,), jnp.float32),
            pltpu.VMEM((_M,), jnp.int32),
            pltpu.VMEM((_T,), jnp.int32),
            pltpu.VMEM((_BATCH,), jnp.float32),
            pltpu.SemaphoreType.DMA,
            pltpu.SemaphoreType.DMA,
            pltpu.SemaphoreType.DMA,
            pltpu.SemaphoreType.DMA,
            pltpu.SemaphoreType.DMA,
        ],
        compiler_params=pltpu.CompilerParams(
            use_tc_tiling_on_sc=True, needs_layout_passes=False),
    )
    return f(x_t, tables_t).T


# speedup vs baseline: 1.2876x; 1.2876x over previous
import jax
import jax.numpy as jnp
from jax import lax
from jax.experimental import pallas as pl
from jax.experimental.pallas import tpu as pltpu
from jax.experimental.pallas import tpu_sc as plsc

_BATCH = 16384
_NUM_FIELDS = 26
_VOCAB = 100000
_EMB_DIM = 32

_NC = 2
_NS = 16
_NW = _NC * _NS
_HB = _BATCH // 2


def _sc_body(x_hbm, table_hbm, out_hbm, row_v, tail_v, idx_v, acc, sem_r,
             sem_t, sem_i):
    s = lax.axis_index("s") * _NC + lax.axis_index("c")

    def field(i, first):
        bnds = (0, 25088, 50176, 75264, 99968)
        cps = [pltpu.async_copy(
                   table_hbm.at[i, s, pl.ds(bnds[k], bnds[k + 1] - bnds[k])],
                   row_v.at[pl.ds(bnds[k], bnds[k + 1] - bnds[k])], sem_r)
               for k in range(4)]
        tail_cp = pltpu.async_copy(
            table_hbm.at[i, s, pl.ds(99968, 32)], tail_v, sem_t)
        idx_cp = pltpu.async_copy(x_hbm.at[i, pl.ds(0, _HB)], idx_v, sem_i)
        for cp in cps:
            cp.wait()
        tail_cp.wait()
        row_v[pl.ds(99968, 16)] = tail_v[pl.ds(0, 16)]
        row_v[pl.ds(99984, 16)] = tail_v[pl.ds(16, 16)]

        def process(half_base):
            def chunk(j, carry):
                v = idx_v[pl.ds(j * 16, 16)]
                g = plsc.load_gather(row_v, [v])
                sl = pl.ds(half_base + j * 16, 16)
                if first:
                    acc[sl] = g
                else:
                    plsc.addupdate(acc.at[sl], g)
                return carry
            lax.fori_loop(0, _HB // 16, chunk, 0, unroll=8)

        idx_cp.wait()
        process(0)
        idx_cp2 = pltpu.async_copy(x_hbm.at[i, pl.ds(_HB, _HB)], idx_v, sem_i)
        idx_cp2.wait()
        process(_HB)

    field(0, True)
    lax.fori_loop(1, _NUM_FIELDS, lambda i, c: (field(i, False), c)[1], 0)
    pltpu.sync_copy(acc, out_hbm.at[s])


def kernel(x, tables):
    x_t = x.T
    tables_t = tables.transpose(0, 2, 1)

    f = pl.kernel(
        _sc_body,
        out_type=jax.ShapeDtypeStruct((_EMB_DIM, _BATCH), jnp.float32),
        mesh=plsc.VectorSubcoreMesh(core_axis_name="c", subcore_axis_name="s"),
        scratch_types=[
            pltpu.VMEM((_VOCAB,), jnp.float32),
            pltpu.VMEM((32,), jnp.float32),
            pltpu.VMEM((_HB,), jnp.int32),
            pltpu.VMEM((_BATCH,), jnp.float32),
            pltpu.SemaphoreType.DMA,
            pltpu.SemaphoreType.DMA,
            pltpu.SemaphoreType.DMA,
        ],
        compiler_params=pltpu.CompilerParams(
            use_tc_tiling_on_sc=True, needs_layout_passes=False),
    )
    out_t = f(x_t, tables_t)
    return out_t.T
